# baseline (speedup 1.0000x reference)
import functools

import jax
import jax.numpy as jnp
from jax import lax
from jax.experimental import pallas as pl
from jax.experimental.pallas import tpu as pltpu
from jax.experimental.pallas import tpu_sc as plsc

N_NODES = 10000
N_EDGES = 320000
D_IN = 128
D_HID = 256
D_OUT = 64

NC = 2
NS = 16
NW = NC * NS
NPAD = 10240
EB1, NB1 = 96, 106
EB2, NB2 = 128, 80
SPT = NPAD // NS
CH = 64

_MESH = dict(core_axis_name="c", subcore_axis_name="s", num_cores=NC,
             num_subcores=NS)


@functools.cache
def _make_degree():
    @functools.partial(
        pl.kernel,
        out_type=jax.ShapeDtypeStruct((NC, NPAD), jnp.float32),
        mesh=plsc.VectorSubcoreMesh(**_MESH),
        scratch_types=[
            pltpu.VMEM((NB2, EB2), jnp.int32),
            pltpu.VMEM((EB2,), jnp.float32),
            pltpu.VMEM((SPT,), jnp.float32),
            pltpu.VMEM_SHARED((NPAD,), jnp.float32),
        ],
    )
    def degree(dst_hbm, out_hbm, idx_v, ones_v, buf_v, acc_sh):
        c = lax.axis_index("c")
        s = lax.axis_index("s")
        tile = c * NS + s
        for i in range(EB2 // 16):
            ones_v[pl.ds(16 * i, 16)] = jnp.ones((16,), jnp.float32)
        for i in range(SPT // 16):
            buf_v[pl.ds(16 * i, 16)] = jnp.zeros((16,), jnp.float32)
        pltpu.sync_copy(buf_v, acc_sh.at[pl.ds(s * SPT, SPT)])
        pltpu.sync_copy(dst_hbm.at[tile], idx_v)
        plsc.subcore_barrier()

        def body(b, carry):
            pltpu.sync_copy(ones_v, acc_sh.at[idx_v.at[b]], add=True)
            return carry

        lax.fori_loop(0, NB2, body, 0)
        plsc.subcore_barrier()
        pltpu.sync_copy(acc_sh.at[pl.ds(s * SPT, SPT)], buf_v)
        pltpu.sync_copy(buf_v, out_hbm.at[c, pl.ds(s * SPT, SPT)])

    return degree


@functools.cache
def _make_rowscatter(D, EB, NB):
    @functools.partial(
        pl.kernel,
        out_type=jax.ShapeDtypeStruct((NC, NPAD, D), jnp.float32),
        mesh=plsc.VectorSubcoreMesh(**_MESH),
        scratch_types=[
            pltpu.VMEM((NB, EB), jnp.int32),
            pltpu.VMEM((NB, EB), jnp.int32),
            pltpu.VMEM((EB, D), jnp.float32),
            pltpu.VMEM((EB, D), jnp.float32),
            pltpu.SemaphoreType.DMA,
            pltpu.SemaphoreType.DMA,
            pltpu.SemaphoreType.DMA,
            pltpu.SemaphoreType.DMA,
            pltpu.VMEM_SHARED((NPAD, D), jnp.float32),
        ],
        compiler_params=pltpu.CompilerParams(use_tc_tiling_on_sc=False),
    )
    def rowscatter(table_hbm, src_hbm, dst_hbm, out_hbm, isv, idv, rows0, rows1,
                   gsem0, gsem1, ssem0, ssem1, acc_sh):
        c = lax.axis_index("c")
        s = lax.axis_index("s")
        tile = c * NS + s
        pltpu.sync_copy(src_hbm.at[tile], isv)
        pltpu.sync_copy(dst_hbm.at[tile], idv)
        pltpu.sync_copy(table_hbm.at[pl.ds(s * SPT, SPT)],
                        acc_sh.at[pl.ds(s * SPT, SPT)])
        plsc.subcore_barrier()

        pltpu.async_copy(table_hbm.at[isv.at[0]], rows0, gsem0)
        pltpu.async_copy(table_hbm.at[isv.at[1]], rows1, gsem1)

        def body(i, carry):
            b0 = 2 * i
            b1 = b0 + 1
            n0 = jnp.where(b0 + 2 >= NB, 0, b0 + 2)
            n1 = jnp.where(b1 + 2 >= NB, 1, b1 + 2)
            pltpu.make_async_copy(table_hbm.at[isv.at[b0]], rows0, gsem0).wait()
            pltpu.async_copy(rows0, acc_sh.at[idv.at[b0]], ssem0, add=True)
            pltpu.make_async_copy(table_hbm.at[isv.at[b1]], rows1, gsem1).wait()
            pltpu.async_copy(rows1, acc_sh.at[idv.at[b1]], ssem1, add=True)
            pltpu.make_async_copy(rows0, acc_sh.at[idv.at[b0]], ssem0).wait()
            pltpu.async_copy(table_hbm.at[isv.at[n0]], rows0, gsem0)
            pltpu.make_async_copy(rows1, acc_sh.at[idv.at[b1]], ssem1).wait()
            pltpu.async_copy(table_hbm.at[isv.at[n1]], rows1, gsem1)
            return carry

        lax.fori_loop(0, NB // 2, body, 0)
        pltpu.make_async_copy(table_hbm.at[isv.at[0]], rows0, gsem0).wait()
        pltpu.make_async_copy(table_hbm.at[isv.at[1]], rows1, gsem1).wait()
        plsc.subcore_barrier()
        pltpu.sync_copy(acc_sh.at[pl.ds(s * SPT, SPT)],
                        out_hbm.at[c, pl.ds(s * SPT, SPT)])

    return rowscatter


def _tc_prescale_body(deg_ref, x_ref, dis_ref, xs_ref):
    deg = deg_ref[:, 0:1] + deg_ref[:, 1:2] + 1.0
    dis = lax.rsqrt(deg)
    dis_b = jnp.broadcast_to(dis, dis_ref.shape)
    dis_ref[...] = dis_b
    xs_ref[...] = x_ref[...] * dis_b


_RB = 1280
_NRB = NPAD // _RB


def _tc_prescale(deg2t, xpad):
    return pl.pallas_call(
        _tc_prescale_body,
        grid=(_NRB,),
        in_specs=[
            pl.BlockSpec((_RB, NC), lambda i: (i, 0)),
            pl.BlockSpec((_RB, D_IN), lambda i: (i, 0)),
        ],
        out_specs=[
            pl.BlockSpec((_RB, D_IN), lambda i: (i, 0)),
            pl.BlockSpec((_RB, D_IN), lambda i: (i, 0)),
        ],
        out_shape=[
            jax.ShapeDtypeStruct((NPAD, D_IN), jnp.float32),
            jax.ShapeDtypeStruct((NPAD, D_IN), jnp.float32),
        ],
    )(deg2t, xpad)


def _tc_mid_body(acc_ref, xs_ref, dis_ref, w1_ref, b1_ref, w2_ref, ts_ref):
    agg = (acc_ref[0] + acc_ref[1] - xs_ref[...]) * dis_ref[...]
    h = jnp.dot(agg, w1_ref[...], preferred_element_type=jnp.float32)
    h = jnp.maximum(h + b1_ref[...], 0.0)
    t = jnp.dot(h, w2_ref[...], preferred_element_type=jnp.float32)
    ts_ref[...] = t * dis_ref[:, :D_OUT]


def _tc_mid(acc, xs, dis128, W1, b1r, W2):
    return pl.pallas_call(
        _tc_mid_body,
        grid=(_NRB,),
        in_specs=[
            pl.BlockSpec((NC, _RB, D_IN), lambda i: (0, i, 0)),
            pl.BlockSpec((_RB, D_IN), lambda i: (i, 0)),
            pl.BlockSpec((_RB, D_IN), lambda i: (i, 0)),
            pl.BlockSpec((D_IN, D_HID), lambda i: (0, 0)),
            pl.BlockSpec((1, D_HID), lambda i: (0, 0)),
            pl.BlockSpec((D_HID, D_OUT), lambda i: (0, 0)),
        ],
        out_specs=pl.BlockSpec((_RB, D_OUT), lambda i: (i, 0)),
        out_shape=jax.ShapeDtypeStruct((NPAD, D_OUT), jnp.float32),
    )(acc, xs, dis128, W1, b1r, W2)


def _tc_final_body(acc_ref, ts_ref, dis_ref, b2_ref, out_ref):
    agg = (acc_ref[0] + acc_ref[1] - ts_ref[...]) * dis_ref[:, :D_OUT]
    out_ref[...] = agg + b2_ref[...]


def _tc_final(acc2, ts, dis128, b2r):
    return pl.pallas_call(
        _tc_final_body,
        grid=(_NRB,),
        in_specs=[
            pl.BlockSpec((NC, _RB, D_OUT), lambda i: (0, i, 0)),
            pl.BlockSpec((_RB, D_OUT), lambda i: (i, 0)),
            pl.BlockSpec((_RB, D_IN), lambda i: (i, 0)),
            pl.BlockSpec((1, D_OUT), lambda i: (0, 0)),
        ],
        out_specs=pl.BlockSpec((_RB, D_OUT), lambda i: (i, 0)),
        out_shape=jax.ShapeDtypeStruct((NPAD, D_OUT), jnp.float32),
    )(acc2, ts, dis128, b2r)


def kernel(x, edge_index, W1, b1, W2, b2):
    x = x.astype(jnp.float32)

    def edge_layout(eb, nb):
        e_pad = NW * nb * eb
        pad_idx = N_NODES + (
            jnp.arange(e_pad - N_EDGES, dtype=jnp.int32) % (NPAD - N_NODES)
        )
        s = jnp.concatenate([edge_index[0].astype(jnp.int32), pad_idx])
        d = jnp.concatenate([edge_index[1].astype(jnp.int32), pad_idx])
        return s.reshape(NW, nb, eb), d.reshape(NW, nb, eb)

    src1, dst1 = edge_layout(EB1, NB1)
    src2, dst2 = edge_layout(EB2, NB2)
    xpad = jnp.pad(x, ((0, NPAD - N_NODES), (0, 0)))

    deg2 = _make_degree()(dst2)
    dis128, xs = _tc_prescale(deg2.T, xpad)
    acc1 = _make_rowscatter(D_IN, EB1, NB1)(xs, src1, dst1)
    ts = _tc_mid(acc1, xs, dis128, W1, b1.reshape(1, D_HID), W2)
    acc2 = _make_rowscatter(D_OUT, EB2, NB2)(ts, src2, dst2)
    out = _tc_final(acc2, ts, dis128, b2.reshape(1, D_OUT))
    return out[:N_NODES]


# speedup vs baseline: 1.1954x; 1.1954x over previous
import functools

import jax
import jax.numpy as jnp
from jax import lax
from jax.experimental import pallas as pl
from jax.experimental.pallas import tpu as pltpu
from jax.experimental.pallas import tpu_sc as plsc

N_NODES = 10000
N_EDGES = 320000
D_IN = 128
D_HID = 256
D_OUT = 64

NC = 2
NS = 16
NW = NC * NS
NPAD = 10240
EB1, NB1 = 96, 106
EB2, NB2 = 128, 80
SPT = NPAD // NS
CH = 64

_MESH = dict(core_axis_name="c", subcore_axis_name="s", num_cores=NC,
             num_subcores=NS)


@functools.cache
def _make_degree():
    @functools.partial(
        pl.kernel,
        out_type=jax.ShapeDtypeStruct((NC, NPAD), jnp.float32),
        mesh=plsc.VectorSubcoreMesh(**_MESH),
        scratch_types=[
            pltpu.VMEM((NB2, EB2), jnp.int32),
            pltpu.VMEM((EB2,), jnp.float32),
            pltpu.VMEM((SPT,), jnp.float32),
            pltpu.VMEM_SHARED((NPAD,), jnp.float32),
        ],
    )
    def degree(dst_hbm, out_hbm, idx_v, ones_v, buf_v, acc_sh):
        c = lax.axis_index("c")
        s = lax.axis_index("s")
        tile = c * NS + s
        for i in range(EB2 // 16):
            ones_v[pl.ds(16 * i, 16)] = jnp.ones((16,), jnp.float32)
        for i in range(SPT // 16):
            buf_v[pl.ds(16 * i, 16)] = jnp.zeros((16,), jnp.float32)
        pltpu.sync_copy(buf_v, acc_sh.at[pl.ds(s * SPT, SPT)])
        pltpu.sync_copy(dst_hbm.at[tile], idx_v)
        plsc.subcore_barrier()

        def body(b, carry):
            pltpu.sync_copy(ones_v, acc_sh.at[idx_v.at[b]], add=True)
            return carry

        lax.fori_loop(0, NB2, body, 0)
        plsc.subcore_barrier()
        pltpu.sync_copy(acc_sh.at[pl.ds(s * SPT, SPT)], buf_v)
        pltpu.sync_copy(buf_v, out_hbm.at[c, pl.ds(s * SPT, SPT)])

    return degree


@functools.cache
def _make_rowscatter(D, EB, NB):
    @functools.partial(
        pl.kernel,
        out_type=jax.ShapeDtypeStruct((NC, NPAD, D), jnp.float32),
        mesh=plsc.VectorSubcoreMesh(**_MESH),
        scratch_types=[
            pltpu.VMEM((NB, EB), jnp.int32),
            pltpu.VMEM((NB, EB), jnp.int32),
            pltpu.VMEM((EB, D), jnp.float32),
            pltpu.VMEM((EB, D), jnp.float32),
            pltpu.SemaphoreType.DMA,
            pltpu.SemaphoreType.DMA,
            pltpu.SemaphoreType.DMA,
            pltpu.SemaphoreType.DMA,
            pltpu.VMEM_SHARED((NPAD, D), jnp.float32),
        ],
        compiler_params=pltpu.CompilerParams(use_tc_tiling_on_sc=False),
    )
    def rowscatter(table_hbm, src_hbm, dst_hbm, out_hbm, isv, idv, rows0, rows1,
                   gsem0, gsem1, ssem0, ssem1, acc_sh):
        c = lax.axis_index("c")
        s = lax.axis_index("s")
        tile = c * NS + s
        pltpu.sync_copy(src_hbm.at[tile], isv)
        pltpu.sync_copy(dst_hbm.at[tile], idv)
        pltpu.sync_copy(table_hbm.at[pl.ds(s * SPT, SPT)],
                        acc_sh.at[pl.ds(s * SPT, SPT)])
        plsc.subcore_barrier()

        pltpu.async_copy(table_hbm.at[isv.at[0]], rows0, gsem0)
        pltpu.async_copy(table_hbm.at[isv.at[1]], rows1, gsem1)

        def body(i, carry):
            b0 = 2 * i
            b1 = b0 + 1
            n0 = jnp.where(b0 + 2 >= NB, 0, b0 + 2)
            n1 = jnp.where(b1 + 2 >= NB, 1, b1 + 2)
            pltpu.make_async_copy(table_hbm.at[isv.at[b0]], rows0, gsem0).wait()
            pltpu.sync_copy(rows0, acc_sh.at[idv.at[b0]], add=True)
            pltpu.async_copy(table_hbm.at[isv.at[n0]], rows0, gsem0)
            pltpu.make_async_copy(table_hbm.at[isv.at[b1]], rows1, gsem1).wait()
            pltpu.sync_copy(rows1, acc_sh.at[idv.at[b1]], add=True)
            pltpu.async_copy(table_hbm.at[isv.at[n1]], rows1, gsem1)
            return carry

        lax.fori_loop(0, NB // 2, body, 0)
        pltpu.make_async_copy(table_hbm.at[isv.at[0]], rows0, gsem0).wait()
        pltpu.make_async_copy(table_hbm.at[isv.at[1]], rows1, gsem1).wait()
        plsc.subcore_barrier()
        pltpu.sync_copy(acc_sh.at[pl.ds(s * SPT, SPT)],
                        out_hbm.at[c, pl.ds(s * SPT, SPT)])

    return rowscatter


def _tc_prescale_body(deg_ref, x_ref, dis_ref, xs_ref):
    deg = deg_ref[:, 0:1] + deg_ref[:, 1:2] + 1.0
    dis = lax.rsqrt(deg)
    dis_b = jnp.broadcast_to(dis, dis_ref.shape)
    dis_ref[...] = dis_b
    xs_ref[...] = x_ref[...] * dis_b


_RB = 1280
_NRB = NPAD // _RB


def _tc_prescale(deg2t, xpad):
    return pl.pallas_call(
        _tc_prescale_body,
        grid=(_NRB,),
        in_specs=[
            pl.BlockSpec((_RB, NC), lambda i: (i, 0)),
            pl.BlockSpec((_RB, D_IN), lambda i: (i, 0)),
        ],
        out_specs=[
            pl.BlockSpec((_RB, D_IN), lambda i: (i, 0)),
            pl.BlockSpec((_RB, D_IN), lambda i: (i, 0)),
        ],
        out_shape=[
            jax.ShapeDtypeStruct((NPAD, D_IN), jnp.float32),
            jax.ShapeDtypeStruct((NPAD, D_IN), jnp.float32),
        ],
    )(deg2t, xpad)


def _tc_mid_body(acc_ref, xs_ref, dis_ref, w1_ref, b1_ref, w2_ref, ts_ref):
    agg = (acc_ref[0] + acc_ref[1] - xs_ref[...]) * dis_ref[...]
    h = jnp.dot(agg, w1_ref[...], preferred_element_type=jnp.float32)
    h = jnp.maximum(h + b1_ref[...], 0.0)
    t = jnp.dot(h, w2_ref[...], preferred_element_type=jnp.float32)
    ts_ref[...] = t * dis_ref[:, :D_OUT]


def _tc_mid(acc, xs, dis128, W1, b1r, W2):
    return pl.pallas_call(
        _tc_mid_body,
        grid=(_NRB,),
        in_specs=[
            pl.BlockSpec((NC, _RB, D_IN), lambda i: (0, i, 0)),
            pl.BlockSpec((_RB, D_IN), lambda i: (i, 0)),
            pl.BlockSpec((_RB, D_IN), lambda i: (i, 0)),
            pl.BlockSpec((D_IN, D_HID), lambda i: (0, 0)),
            pl.BlockSpec((1, D_HID), lambda i: (0, 0)),
            pl.BlockSpec((D_HID, D_OUT), lambda i: (0, 0)),
        ],
        out_specs=pl.BlockSpec((_RB, D_OUT), lambda i: (i, 0)),
        out_shape=jax.ShapeDtypeStruct((NPAD, D_OUT), jnp.float32),
    )(acc, xs, dis128, W1, b1r, W2)


def _tc_final_body(acc_ref, ts_ref, dis_ref, b2_ref, out_ref):
    agg = (acc_ref[0] + acc_ref[1] - ts_ref[...]) * dis_ref[:, :D_OUT]
    out_ref[...] = agg + b2_ref[...]


def _tc_final(acc2, ts, dis128, b2r):
    return pl.pallas_call(
        _tc_final_body,
        grid=(_NRB,),
        in_specs=[
            pl.BlockSpec((NC, _RB, D_OUT), lambda i: (0, i, 0)),
            pl.BlockSpec((_RB, D_OUT), lambda i: (i, 0)),
            pl.BlockSpec((_RB, D_IN), lambda i: (i, 0)),
            pl.BlockSpec((1, D_OUT), lambda i: (0, 0)),
        ],
        out_specs=pl.BlockSpec((_RB, D_OUT), lambda i: (i, 0)),
        out_shape=jax.ShapeDtypeStruct((NPAD, D_OUT), jnp.float32),
    )(acc2, ts, dis128, b2r)


def kernel(x, edge_index, W1, b1, W2, b2):
    x = x.astype(jnp.float32)

    def edge_layout(eb, nb):
        e_pad = NW * nb * eb
        pad_idx = N_NODES + (
            jnp.arange(e_pad - N_EDGES, dtype=jnp.int32) % (NPAD - N_NODES)
        )
        s = jnp.concatenate([edge_index[0].astype(jnp.int32), pad_idx])
        d = jnp.concatenate([edge_index[1].astype(jnp.int32), pad_idx])
        return s.reshape(NW, nb, eb), d.reshape(NW, nb, eb)

    src1, dst1 = edge_layout(EB1, NB1)
    src2, dst2 = edge_layout(EB2, NB2)
    xpad = jnp.pad(x, ((0, NPAD - N_NODES), (0, 0)))

    deg2 = _make_degree()(dst2)
    dis128, xs = _tc_prescale(deg2.T, xpad)
    acc1 = _make_rowscatter(D_IN, EB1, NB1)(xs, src1, dst1)
    ts = _tc_mid(acc1, xs, dis128, W1, b1.reshape(1, D_HID), W2)
    acc2 = _make_rowscatter(D_OUT, EB2, NB2)(ts, src2, dst2)
    out = _tc_final(acc2, ts, dis128, b2.reshape(1, D_OUT))
    return out[:N_NODES]


# speedup vs baseline: 1.2223x; 1.0225x over previous
import functools

import jax
import jax.numpy as jnp
from jax import lax
from jax.experimental import pallas as pl
from jax.experimental.pallas import tpu as pltpu
from jax.experimental.pallas import tpu_sc as plsc

N_NODES = 10000
N_EDGES = 320000
D_IN = 128
D_HID = 256
D_OUT = 64

NC = 2
NS = 16
NW = NC * NS
NPAD = 10240
EB, NB = 128, 80
SPT = NPAD // NS

_MESH = dict(core_axis_name="c", subcore_axis_name="s", num_cores=NC,
             num_subcores=NS)


@functools.cache
def _make_degree():
    @functools.partial(
        pl.kernel,
        out_type=jax.ShapeDtypeStruct((NC, NPAD), jnp.float32),
        mesh=plsc.VectorSubcoreMesh(**_MESH),
        scratch_types=[
            pltpu.VMEM((NB, EB), jnp.int32),
            pltpu.VMEM((EB,), jnp.float32),
            pltpu.VMEM((SPT,), jnp.float32),
            pltpu.VMEM_SHARED((NPAD,), jnp.float32),
        ],
    )
    def degree(dst_hbm, out_hbm, idx_v, ones_v, buf_v, acc_sh):
        c = lax.axis_index("c")
        s = lax.axis_index("s")
        tile = c * NS + s
        for i in range(EB // 16):
            ones_v[pl.ds(16 * i, 16)] = jnp.ones((16,), jnp.float32)
        for i in range(SPT // 16):
            buf_v[pl.ds(16 * i, 16)] = jnp.zeros((16,), jnp.float32)
        pltpu.sync_copy(buf_v, acc_sh.at[pl.ds(s * SPT, SPT)])
        pltpu.sync_copy(dst_hbm.at[tile], idx_v)
        plsc.subcore_barrier()

        def body(b, carry):
            pltpu.sync_copy(ones_v, acc_sh.at[idx_v.at[b]], add=True)
            return carry

        lax.fori_loop(0, NB, body, 0)
        plsc.subcore_barrier()
        pltpu.sync_copy(acc_sh.at[pl.ds(s * SPT, SPT)], buf_v)
        pltpu.sync_copy(buf_v, out_hbm.at[c, pl.ds(s * SPT, SPT)])

    return degree


@functools.cache
def _make_rowscatter(D):
    @functools.partial(
        pl.kernel,
        out_type=jax.ShapeDtypeStruct((NC, NPAD, D), jnp.float32),
        mesh=plsc.VectorSubcoreMesh(**_MESH),
        scratch_types=[
            pltpu.VMEM((NB, EB), jnp.int32),
            pltpu.VMEM((EB,), jnp.int32),
            pltpu.VMEM((EB,), jnp.int32),
            pltpu.VMEM((EB,), jnp.int32),
            pltpu.VMEM((EB,), jnp.int32),
            pltpu.VMEM((EB, D), jnp.float32),
            pltpu.VMEM((EB, D), jnp.float32),
            pltpu.SemaphoreType.DMA,
            pltpu.SemaphoreType.DMA,
            pltpu.VMEM_SHARED((NPAD, D), jnp.float32),
        ],
        compiler_params=pltpu.CompilerParams(use_tc_tiling_on_sc=False),
    )
    def rowscatter(table_hbm, pk_hbm, out_hbm, pk, isb0, idb0, isb1, idb1,
                   rows0, rows1, gsem0, gsem1, acc_sh):
        c = lax.axis_index("c")
        s = lax.axis_index("s")
        tile = c * NS + s
        pltpu.sync_copy(pk_hbm.at[tile], pk)
        pltpu.sync_copy(table_hbm.at[pl.ds(s * SPT, SPT)],
                        acc_sh.at[pl.ds(s * SPT, SPT)])
        plsc.subcore_barrier()

        def unpack(b, isb, idb):
            for k in range(EB // 16):
                w = pk[b, pl.ds(16 * k, 16)]
                isb[pl.ds(16 * k, 16)] = w >> 16
                idb[pl.ds(16 * k, 16)] = w & 0xFFFF

        unpack(0, isb0, idb0)
        pltpu.async_copy(table_hbm.at[isb0], rows0, gsem0)
        unpack(1, isb1, idb1)
        pltpu.async_copy(table_hbm.at[isb1], rows1, gsem1)

        def body(i, carry):
            b0 = 2 * i
            b1 = b0 + 1
            n0 = jnp.where(b0 + 2 >= NB, 0, b0 + 2)
            n1 = jnp.where(b1 + 2 >= NB, 1, b1 + 2)
            pltpu.make_async_copy(table_hbm.at[isb0], rows0, gsem0).wait()
            pltpu.sync_copy(rows0, acc_sh.at[idb0], add=True)
            unpack(n0, isb0, idb0)
            pltpu.async_copy(table_hbm.at[isb0], rows0, gsem0)
            pltpu.make_async_copy(table_hbm.at[isb1], rows1, gsem1).wait()
            pltpu.sync_copy(rows1, acc_sh.at[idb1], add=True)
            unpack(n1, isb1, idb1)
            pltpu.async_copy(table_hbm.at[isb1], rows1, gsem1)
            return carry

        lax.fori_loop(0, NB // 2, body, 0)
        pltpu.make_async_copy(table_hbm.at[isb0], rows0, gsem0).wait()
        pltpu.make_async_copy(table_hbm.at[isb1], rows1, gsem1).wait()
        plsc.subcore_barrier()
        pltpu.sync_copy(acc_sh.at[pl.ds(s * SPT, SPT)],
                        out_hbm.at[c, pl.ds(s * SPT, SPT)])

    return rowscatter


def _tc_prescale_body(deg_ref, x_ref, dis_ref, xs_ref):
    deg = deg_ref[:, 0:1] + deg_ref[:, 1:2] + 1.0
    dis = lax.rsqrt(deg)
    dis_b = jnp.broadcast_to(dis, dis_ref.shape)
    dis_ref[...] = dis_b
    xs_ref[...] = x_ref[...] * dis_b


_RB = 1280
_NRB = NPAD // _RB


def _tc_prescale(deg2t, xpad):
    return pl.pallas_call(
        _tc_prescale_body,
        grid=(_NRB,),
        in_specs=[
            pl.BlockSpec((_RB, NC), lambda i: (i, 0)),
            pl.BlockSpec((_RB, D_IN), lambda i: (i, 0)),
        ],
        out_specs=[
            pl.BlockSpec((_RB, D_IN), lambda i: (i, 0)),
            pl.BlockSpec((_RB, D_IN), lambda i: (i, 0)),
        ],
        out_shape=[
            jax.ShapeDtypeStruct((NPAD, D_IN), jnp.float32),
            jax.ShapeDtypeStruct((NPAD, D_IN), jnp.float32),
        ],
    )(deg2t, xpad)


def _tc_mid_body(acc_ref, xs_ref, dis_ref, w1_ref, b1_ref, w2_ref, ts_ref):
    agg = (acc_ref[0] + acc_ref[1] - xs_ref[...]) * dis_ref[...]
    h = jnp.dot(agg, w1_ref[...], preferred_element_type=jnp.float32)
    h = jnp.maximum(h + b1_ref[...], 0.0)
    t = jnp.dot(h, w2_ref[...], preferred_element_type=jnp.float32)
    ts_ref[...] = t * dis_ref[:, :D_OUT]


def _tc_mid(acc, xs, dis128, W1, b1r, W2):
    return pl.pallas_call(
        _tc_mid_body,
        grid=(_NRB,),
        in_specs=[
            pl.BlockSpec((NC, _RB, D_IN), lambda i: (0, i, 0)),
            pl.BlockSpec((_RB, D_IN), lambda i: (i, 0)),
            pl.BlockSpec((_RB, D_IN), lambda i: (i, 0)),
            pl.BlockSpec((D_IN, D_HID), lambda i: (0, 0)),
            pl.BlockSpec((1, D_HID), lambda i: (0, 0)),
            pl.BlockSpec((D_HID, D_OUT), lambda i: (0, 0)),
        ],
        out_specs=pl.BlockSpec((_RB, D_OUT), lambda i: (i, 0)),
        out_shape=jax.ShapeDtypeStruct((NPAD, D_OUT), jnp.float32),
    )(acc, xs, dis128, W1, b1r, W2)


def _tc_final_body(acc_ref, ts_ref, dis_ref, b2_ref, out_ref):
    agg = (acc_ref[0] + acc_ref[1] - ts_ref[...]) * dis_ref[:, :D_OUT]
    out_ref[...] = agg + b2_ref[...]


def _tc_final(acc2, ts, dis128, b2r):
    return pl.pallas_call(
        _tc_final_body,
        grid=(_NRB,),
        in_specs=[
            pl.BlockSpec((NC, _RB, D_OUT), lambda i: (0, i, 0)),
            pl.BlockSpec((_RB, D_OUT), lambda i: (i, 0)),
            pl.BlockSpec((_RB, D_IN), lambda i: (i, 0)),
            pl.BlockSpec((1, D_OUT), lambda i: (0, 0)),
        ],
        out_specs=pl.BlockSpec((_RB, D_OUT), lambda i: (i, 0)),
        out_shape=jax.ShapeDtypeStruct((NPAD, D_OUT), jnp.float32),
    )(acc2, ts, dis128, b2r)


def kernel(x, edge_index, W1, b1, W2, b2):
    x = x.astype(jnp.float32)

    def edge_layout(eb, nb):
        e_pad = NW * nb * eb
        pad_idx = N_NODES + (
            jnp.arange(e_pad - N_EDGES, dtype=jnp.int32) % (NPAD - N_NODES)
        )
        s = jnp.concatenate([edge_index[0].astype(jnp.int32), pad_idx])
        d = jnp.concatenate([edge_index[1].astype(jnp.int32), pad_idx])
        return s.reshape(NW, nb, eb), d.reshape(NW, nb, eb)

    src, dst = edge_layout(EB, NB)
    packed = (src << 16) | dst
    xpad = jnp.pad(x, ((0, NPAD - N_NODES), (0, 0)))

    deg2 = _make_degree()(dst)
    dis128, xs = _tc_prescale(deg2.T, xpad)
    acc1 = _make_rowscatter(D_IN)(xs, packed)
    ts = _tc_mid(acc1, xs, dis128, W1, b1.reshape(1, D_HID), W2)
    acc2 = _make_rowscatter(D_OUT)(ts, packed)
    out = _tc_final(acc2, ts, dis128, b2.reshape(1, D_OUT))
    return out[:N_NODES]


# speedup vs baseline: 1.2337x; 1.0093x over previous
import functools

import jax
import jax.numpy as jnp
from jax import lax
from jax.experimental import pallas as pl
from jax.experimental.pallas import tpu as pltpu
from jax.experimental.pallas import tpu_sc as plsc

N_NODES = 10000
N_EDGES = 320000
D_IN = 128
D_HID = 256
D_OUT = 64

NC = 2
NS = 16
NW = NC * NS
NPAD = 10240
EPT = N_EDGES // NW
EB = 128
NFB = EPT // EB
TB = EPT - NFB * EB
SPT = NPAD // NS
LSPT = N_NODES - (NS - 1) * SPT

_MESH = dict(core_axis_name="c", subcore_axis_name="s", num_cores=NC,
             num_subcores=NS)


def _unpack(pk_v, base, n, isb, idb):
    for k in range(n // 16):
        w = pk_v[pl.ds(base + 16 * k, 16)]
        isb[pl.ds(16 * k, 16)] = w >> 16
        idb[pl.ds(16 * k, 16)] = w & 0xFFFF


@functools.cache
def _make_degree():
    @functools.partial(
        pl.kernel,
        out_type=jax.ShapeDtypeStruct((NC, N_NODES), jnp.float32),
        mesh=plsc.VectorSubcoreMesh(**_MESH),
        scratch_types=[
            pltpu.VMEM((EPT,), jnp.int32),
            pltpu.VMEM((EB,), jnp.int32),
            pltpu.VMEM((EB,), jnp.int32),
            pltpu.VMEM((TB,), jnp.int32),
            pltpu.VMEM((TB,), jnp.int32),
            pltpu.VMEM((EB,), jnp.float32),
            pltpu.VMEM((SPT,), jnp.float32),
            pltpu.VMEM_SHARED((NPAD,), jnp.float32),
        ],
        compiler_params=pltpu.CompilerParams(use_tc_tiling_on_sc=False),
    )
    def degree(pk_hbm, out_hbm, pk_v, isb, idb, ist, idt, ones_v, buf_v,
               acc_sh):
        c = lax.axis_index("c")
        s = lax.axis_index("s")
        tile = c * NS + s
        for i in range(EB // 16):
            ones_v[pl.ds(16 * i, 16)] = jnp.ones((16,), jnp.float32)
        for i in range(SPT // 16):
            buf_v[pl.ds(16 * i, 16)] = jnp.zeros((16,), jnp.float32)
        pltpu.sync_copy(buf_v, acc_sh.at[pl.ds(s * SPT, SPT)])
        pltpu.sync_copy(pk_hbm.at[pl.ds(tile * EPT, EPT)], pk_v)
        plsc.subcore_barrier()

        def body(b, carry):
            _unpack(pk_v, b * EB, EB, isb, idb)
            pltpu.sync_copy(ones_v, acc_sh.at[idb], add=True)
            return carry

        lax.fori_loop(0, NFB, body, 0)
        _unpack(pk_v, NFB * EB, TB, ist, idt)
        pltpu.sync_copy(ones_v.at[pl.ds(0, TB)], acc_sh.at[idt], add=True)
        plsc.subcore_barrier()

        @pl.when(s < NS - 1)
        def _():
            pltpu.sync_copy(acc_sh.at[pl.ds(s * SPT, SPT)], buf_v)
            pltpu.sync_copy(buf_v, out_hbm.at[c, pl.ds(s * SPT, SPT)])

        @pl.when(s == NS - 1)
        def _():
            pltpu.sync_copy(acc_sh.at[pl.ds(s * SPT, LSPT)],
                            buf_v.at[pl.ds(0, LSPT)])
            pltpu.sync_copy(buf_v.at[pl.ds(0, LSPT)],
                            out_hbm.at[c, pl.ds(s * SPT, LSPT)])

    return degree


@functools.cache
def _make_rowscatter(D):
    @functools.partial(
        pl.kernel,
        out_type=jax.ShapeDtypeStruct((NC, N_NODES, D), jnp.float32),
        mesh=plsc.VectorSubcoreMesh(**_MESH),
        scratch_types=[
            pltpu.VMEM((EPT,), jnp.int32),
            pltpu.VMEM((EB,), jnp.int32),
            pltpu.VMEM((EB,), jnp.int32),
            pltpu.VMEM((EB,), jnp.int32),
            pltpu.VMEM((EB,), jnp.int32),
            pltpu.VMEM((TB,), jnp.int32),
            pltpu.VMEM((TB,), jnp.int32),
            pltpu.VMEM((EB, D), jnp.float32),
            pltpu.VMEM((EB, D), jnp.float32),
            pltpu.SemaphoreType.DMA,
            pltpu.SemaphoreType.DMA,
            pltpu.VMEM_SHARED((NPAD, D), jnp.float32),
        ],
        compiler_params=pltpu.CompilerParams(use_tc_tiling_on_sc=False),
    )
    def rowscatter(table_hbm, pk_hbm, out_hbm, pk_v, isb0, idb0, isb1, idb1,
                   ist, idt, rows0, rows1, gsem0, gsem1, acc_sh):
        c = lax.axis_index("c")
        s = lax.axis_index("s")
        tile = c * NS + s
        pltpu.sync_copy(pk_hbm.at[pl.ds(tile * EPT, EPT)], pk_v)
        @pl.when(s < NS - 1)
        def _():
            pltpu.sync_copy(table_hbm.at[pl.ds(s * SPT, SPT)],
                            acc_sh.at[pl.ds(s * SPT, SPT)])

        @pl.when(s == NS - 1)
        def _():
            pltpu.sync_copy(table_hbm.at[pl.ds(s * SPT, LSPT)],
                            acc_sh.at[pl.ds(s * SPT, LSPT)])

        plsc.subcore_barrier()

        _unpack(pk_v, 0, EB, isb0, idb0)
        pltpu.async_copy(table_hbm.at[isb0], rows0, gsem0)
        _unpack(pk_v, EB, EB, isb1, idb1)
        pltpu.async_copy(table_hbm.at[isb1], rows1, gsem1)

        def body(i, carry):
            b0 = 2 * i
            b1 = b0 + 1
            n0 = jnp.where(b0 + 2 >= NFB, 0, b0 + 2)
            n1 = jnp.where(b1 + 2 >= NFB, 1, b1 + 2)
            pltpu.make_async_copy(table_hbm.at[isb0], rows0, gsem0).wait()
            pltpu.sync_copy(rows0, acc_sh.at[idb0], add=True)
            _unpack(pk_v, n0 * EB, EB, isb0, idb0)
            pltpu.async_copy(table_hbm.at[isb0], rows0, gsem0)
            pltpu.make_async_copy(table_hbm.at[isb1], rows1, gsem1).wait()
            pltpu.sync_copy(rows1, acc_sh.at[idb1], add=True)
            _unpack(pk_v, n1 * EB, EB, isb1, idb1)
            pltpu.async_copy(table_hbm.at[isb1], rows1, gsem1)
            return carry

        lax.fori_loop(0, NFB // 2, body, 0)
        pltpu.make_async_copy(table_hbm.at[isb0], rows0, gsem0).wait()
        pltpu.make_async_copy(table_hbm.at[isb1], rows1, gsem1).wait()
        _unpack(pk_v, NFB * EB, TB, ist, idt)
        pltpu.sync_copy(table_hbm.at[ist], rows0.at[pl.ds(0, TB)])
        pltpu.sync_copy(rows0.at[pl.ds(0, TB)], acc_sh.at[idt], add=True)
        plsc.subcore_barrier()

        @pl.when(s < NS - 1)
        def _():
            pltpu.sync_copy(acc_sh.at[pl.ds(s * SPT, SPT)],
                            out_hbm.at[c, pl.ds(s * SPT, SPT)])

        @pl.when(s == NS - 1)
        def _():
            pltpu.sync_copy(acc_sh.at[pl.ds(s * SPT, LSPT)],
                            out_hbm.at[c, pl.ds(s * SPT, LSPT)])

    return rowscatter


def _tc_prescale_body(deg_ref, x_ref, dis_ref, xs_ref):
    deg = deg_ref[:, 0:1] + deg_ref[:, 1:2] + 1.0
    dis = lax.rsqrt(deg)
    dis_b = jnp.broadcast_to(dis, dis_ref.shape)
    dis_ref[...] = dis_b
    xs_ref[...] = x_ref[...] * dis_b


_RB = 2000
_NRB = N_NODES // _RB


def _tc_prescale(deg2t, x):
    return pl.pallas_call(
        _tc_prescale_body,
        grid=(_NRB,),
        in_specs=[
            pl.BlockSpec((_RB, NC), lambda i: (i, 0)),
            pl.BlockSpec((_RB, D_IN), lambda i: (i, 0)),
        ],
        out_specs=[
            pl.BlockSpec((_RB, D_IN), lambda i: (i, 0)),
            pl.BlockSpec((_RB, D_IN), lambda i: (i, 0)),
        ],
        out_shape=[
            jax.ShapeDtypeStruct((N_NODES, D_IN), jnp.float32),
            jax.ShapeDtypeStruct((N_NODES, D_IN), jnp.float32),
        ],
    )(deg2t, x)


def _tc_mid_body(acc_ref, xs_ref, dis_ref, w1_ref, b1_ref, w2_ref, ts_ref):
    agg = (acc_ref[0] + acc_ref[1] - xs_ref[...]) * dis_ref[...]
    h = jnp.dot(agg, w1_ref[...], preferred_element_type=jnp.float32)
    h = jnp.maximum(h + b1_ref[...], 0.0)
    t = jnp.dot(h, w2_ref[...], preferred_element_type=jnp.float32)
    ts_ref[...] = t * dis_ref[:, :D_OUT]


def _tc_mid(acc, xs, dis128, W1, b1r, W2):
    return pl.pallas_call(
        _tc_mid_body,
        grid=(_NRB,),
        in_specs=[
            pl.BlockSpec((NC, _RB, D_IN), lambda i: (0, i, 0)),
            pl.BlockSpec((_RB, D_IN), lambda i: (i, 0)),
            pl.BlockSpec((_RB, D_IN), lambda i: (i, 0)),
            pl.BlockSpec((D_IN, D_HID), lambda i: (0, 0)),
            pl.BlockSpec((1, D_HID), lambda i: (0, 0)),
            pl.BlockSpec((D_HID, D_OUT), lambda i: (0, 0)),
        ],
        out_specs=pl.BlockSpec((_RB, D_OUT), lambda i: (i, 0)),
        out_shape=jax.ShapeDtypeStruct((N_NODES, D_OUT), jnp.float32),
    )(acc, xs, dis128, W1, b1r, W2)


def _tc_final_body(acc_ref, ts_ref, dis_ref, b2_ref, out_ref):
    agg = (acc_ref[0] + acc_ref[1] - ts_ref[...]) * dis_ref[:, :D_OUT]
    out_ref[...] = agg + b2_ref[...]


def _tc_final(acc2, ts, dis128, b2r):
    return pl.pallas_call(
        _tc_final_body,
        grid=(_NRB,),
        in_specs=[
            pl.BlockSpec((NC, _RB, D_OUT), lambda i: (0, i, 0)),
            pl.BlockSpec((_RB, D_OUT), lambda i: (i, 0)),
            pl.BlockSpec((_RB, D_IN), lambda i: (i, 0)),
            pl.BlockSpec((1, D_OUT), lambda i: (0, 0)),
        ],
        out_specs=pl.BlockSpec((_RB, D_OUT), lambda i: (i, 0)),
        out_shape=jax.ShapeDtypeStruct((N_NODES, D_OUT), jnp.float32),
    )(acc2, ts, dis128, b2r)


def kernel(x, edge_index, W1, b1, W2, b2):
    x = x.astype(jnp.float32)
    ei = edge_index.astype(jnp.int32)
    packed = (ei[0] << 16) | ei[1]

    deg2 = _make_degree()(packed)
    dis128, xs = _tc_prescale(deg2.T, x)
    acc1 = _make_rowscatter(D_IN)(xs, packed)
    ts = _tc_mid(acc1, xs, dis128, W1, b1.reshape(1, D_HID), W2)
    acc2 = _make_rowscatter(D_OUT)(ts, packed)
    out = _tc_final(acc2, ts, dis128, b2.reshape(1, D_OUT))
    return out


# speedup vs baseline: 1.2353x; 1.0013x over previous
import functools

import jax
import jax.numpy as jnp
from jax import lax
from jax.experimental import pallas as pl
from jax.experimental.pallas import tpu as pltpu
from jax.experimental.pallas import tpu_sc as plsc

N_NODES = 10000
N_EDGES = 320000
D_IN = 128
D_HID = 256
D_OUT = 64

NC = 2
NS = 16
NW = NC * NS
NPAD = 10240
EPT = N_EDGES // NW
EB = 128
NFB = EPT // EB
TB = EPT - NFB * EB
SPT = NPAD // NS
LSPT = N_NODES - (NS - 1) * SPT

_MESH = dict(core_axis_name="c", subcore_axis_name="s", num_cores=NC,
             num_subcores=NS)


def _unpack(pk_v, base, n, isb, idb):
    for k in range(n // 16):
        w = pk_v[pl.ds(base + 16 * k, 16)]
        isb[pl.ds(16 * k, 16)] = w >> 16
        idb[pl.ds(16 * k, 16)] = w & 0xFFFF


@functools.cache
def _make_degree():
    @functools.partial(
        pl.kernel,
        out_type=jax.ShapeDtypeStruct((NC, N_NODES), jnp.float32),
        mesh=plsc.VectorSubcoreMesh(**_MESH),
        scratch_types=[
            pltpu.VMEM((EPT,), jnp.int32),
            pltpu.VMEM((EB,), jnp.int32),
            pltpu.VMEM((EB,), jnp.int32),
            pltpu.VMEM((TB,), jnp.int32),
            pltpu.VMEM((TB,), jnp.int32),
            pltpu.VMEM((EB,), jnp.float32),
            pltpu.VMEM((SPT,), jnp.float32),
            pltpu.VMEM_SHARED((NPAD,), jnp.float32),
        ],
        compiler_params=pltpu.CompilerParams(use_tc_tiling_on_sc=False),
    )
    def degree(pk_hbm, out_hbm, pk_v, isb, idb, ist, idt, ones_v, buf_v,
               acc_sh):
        c = lax.axis_index("c")
        s = lax.axis_index("s")
        tile = c * NS + s
        for i in range(EB // 16):
            ones_v[pl.ds(16 * i, 16)] = jnp.ones((16,), jnp.float32)
        for i in range(SPT // 16):
            buf_v[pl.ds(16 * i, 16)] = jnp.zeros((16,), jnp.float32)
        pltpu.sync_copy(buf_v, acc_sh.at[pl.ds(s * SPT, SPT)])
        pltpu.sync_copy(pk_hbm.at[pl.ds(tile * EPT, EPT)], pk_v)
        plsc.subcore_barrier()

        def body(b, carry):
            _unpack(pk_v, b * EB, EB, isb, idb)
            pltpu.sync_copy(ones_v, acc_sh.at[idb], add=True)
            return carry

        lax.fori_loop(0, NFB, body, 0)
        _unpack(pk_v, NFB * EB, TB, ist, idt)
        pltpu.sync_copy(ones_v.at[pl.ds(0, TB)], acc_sh.at[idt], add=True)
        plsc.subcore_barrier()

        @pl.when(s < NS - 1)
        def _():
            pltpu.sync_copy(acc_sh.at[pl.ds(s * SPT, SPT)], buf_v)
            pltpu.sync_copy(buf_v, out_hbm.at[c, pl.ds(s * SPT, SPT)])

        @pl.when(s == NS - 1)
        def _():
            pltpu.sync_copy(acc_sh.at[pl.ds(s * SPT, LSPT)],
                            buf_v.at[pl.ds(0, LSPT)])
            pltpu.sync_copy(buf_v.at[pl.ds(0, LSPT)],
                            out_hbm.at[c, pl.ds(s * SPT, LSPT)])

    return degree


@functools.cache
def _make_rowscatter(D):
    @functools.partial(
        pl.kernel,
        out_type=jax.ShapeDtypeStruct((NC, N_NODES, D), jnp.float32),
        mesh=plsc.VectorSubcoreMesh(**_MESH),
        scratch_types=[
            pltpu.VMEM((EPT,), jnp.int32),
            pltpu.VMEM((EB,), jnp.int32),
            pltpu.VMEM((EB,), jnp.int32),
            pltpu.VMEM((EB,), jnp.int32),
            pltpu.VMEM((EB,), jnp.int32),
            pltpu.VMEM((TB,), jnp.int32),
            pltpu.VMEM((TB,), jnp.int32),
            pltpu.VMEM((EB, D), jnp.float32),
            pltpu.VMEM((EB, D), jnp.float32),
            pltpu.SemaphoreType.DMA,
            pltpu.SemaphoreType.DMA,
            pltpu.VMEM_SHARED((NPAD, D), jnp.float32),
        ],
        compiler_params=pltpu.CompilerParams(use_tc_tiling_on_sc=(D == D_IN)),
    )
    def rowscatter(table_hbm, pk_hbm, out_hbm, pk_v, isb0, idb0, isb1, idb1,
                   ist, idt, rows0, rows1, gsem0, gsem1, acc_sh):
        c = lax.axis_index("c")
        s = lax.axis_index("s")
        tile = c * NS + s
        pltpu.sync_copy(pk_hbm.at[pl.ds(tile * EPT, EPT)], pk_v)
        @pl.when(s < NS - 1)
        def _():
            pltpu.sync_copy(table_hbm.at[pl.ds(s * SPT, SPT)],
                            acc_sh.at[pl.ds(s * SPT, SPT)])

        @pl.when(s == NS - 1)
        def _():
            pltpu.sync_copy(table_hbm.at[pl.ds(s * SPT, LSPT)],
                            acc_sh.at[pl.ds(s * SPT, LSPT)])

        plsc.subcore_barrier()

        _unpack(pk_v, 0, EB, isb0, idb0)
        pltpu.async_copy(table_hbm.at[isb0], rows0, gsem0)
        _unpack(pk_v, EB, EB, isb1, idb1)
        pltpu.async_copy(table_hbm.at[isb1], rows1, gsem1)

        def body(i, carry):
            b0 = 2 * i
            b1 = b0 + 1
            n0 = jnp.where(b0 + 2 >= NFB, 0, b0 + 2)
            n1 = jnp.where(b1 + 2 >= NFB, 1, b1 + 2)
            pltpu.make_async_copy(table_hbm.at[isb0], rows0, gsem0).wait()
            pltpu.sync_copy(rows0, acc_sh.at[idb0], add=True)
            _unpack(pk_v, n0 * EB, EB, isb0, idb0)
            pltpu.async_copy(table_hbm.at[isb0], rows0, gsem0)
            pltpu.make_async_copy(table_hbm.at[isb1], rows1, gsem1).wait()
            pltpu.sync_copy(rows1, acc_sh.at[idb1], add=True)
            _unpack(pk_v, n1 * EB, EB, isb1, idb1)
            pltpu.async_copy(table_hbm.at[isb1], rows1, gsem1)
            return carry

        lax.fori_loop(0, NFB // 2, body, 0)
        pltpu.make_async_copy(table_hbm.at[isb0], rows0, gsem0).wait()
        pltpu.make_async_copy(table_hbm.at[isb1], rows1, gsem1).wait()
        _unpack(pk_v, NFB * EB, TB, ist, idt)
        pltpu.sync_copy(table_hbm.at[ist], rows0.at[pl.ds(0, TB)])
        pltpu.sync_copy(rows0.at[pl.ds(0, TB)], acc_sh.at[idt], add=True)
        plsc.subcore_barrier()

        @pl.when(s < NS - 1)
        def _():
            pltpu.sync_copy(acc_sh.at[pl.ds(s * SPT, SPT)],
                            out_hbm.at[c, pl.ds(s * SPT, SPT)])

        @pl.when(s == NS - 1)
        def _():
            pltpu.sync_copy(acc_sh.at[pl.ds(s * SPT, LSPT)],
                            out_hbm.at[c, pl.ds(s * SPT, LSPT)])

    return rowscatter


def _tc_prescale_body(deg_ref, x_ref, dis_ref, xs_ref):
    deg = deg_ref[:, 0:1] + deg_ref[:, 1:2] + 1.0
    dis = lax.rsqrt(deg)
    dis_b = jnp.broadcast_to(dis, dis_ref.shape)
    dis_ref[...] = dis_b
    xs_ref[...] = x_ref[...] * dis_b


_RB = 2000
_NRB = N_NODES // _RB


def _tc_prescale(deg2t, x):
    return pl.pallas_call(
        _tc_prescale_body,
        grid=(_NRB,),
        in_specs=[
            pl.BlockSpec((_RB, NC), lambda i: (i, 0)),
            pl.BlockSpec((_RB, D_IN), lambda i: (i, 0)),
        ],
        out_specs=[
            pl.BlockSpec((_RB, D_IN), lambda i: (i, 0)),
            pl.BlockSpec((_RB, D_IN), lambda i: (i, 0)),
        ],
        out_shape=[
            jax.ShapeDtypeStruct((N_NODES, D_IN), jnp.float32),
            jax.ShapeDtypeStruct((N_NODES, D_IN), jnp.float32),
        ],
    )(deg2t, x)


def _tc_mid_body(acc_ref, xs_ref, dis_ref, w1_ref, b1_ref, w2_ref, ts_ref):
    agg = (acc_ref[0] + acc_ref[1] - xs_ref[...]) * dis_ref[...]
    h = jnp.dot(agg, w1_ref[...], preferred_element_type=jnp.float32)
    h = jnp.maximum(h + b1_ref[...], 0.0)
    t = jnp.dot(h, w2_ref[...], preferred_element_type=jnp.float32)
    ts_ref[...] = t * dis_ref[:, :D_OUT]


def _tc_mid(acc, xs, dis128, W1, b1r, W2):
    return pl.pallas_call(
        _tc_mid_body,
        grid=(_NRB,),
        in_specs=[
            pl.BlockSpec((NC, _RB, D_IN), lambda i: (0, i, 0)),
            pl.BlockSpec((_RB, D_IN), lambda i: (i, 0)),
            pl.BlockSpec((_RB, D_IN), lambda i: (i, 0)),
            pl.BlockSpec((D_IN, D_HID), lambda i: (0, 0)),
            pl.BlockSpec((1, D_HID), lambda i: (0, 0)),
            pl.BlockSpec((D_HID, D_OUT), lambda i: (0, 0)),
        ],
        out_specs=pl.BlockSpec((_RB, D_OUT), lambda i: (i, 0)),
        out_shape=jax.ShapeDtypeStruct((N_NODES, D_OUT), jnp.float32),
    )(acc, xs, dis128, W1, b1r, W2)


def _tc_final_body(acc_ref, ts_ref, dis_ref, b2_ref, out_ref):
    agg = (acc_ref[0] + acc_ref[1] - ts_ref[...]) * dis_ref[:, :D_OUT]
    out_ref[...] = agg + b2_ref[...]


def _tc_final(acc2, ts, dis128, b2r):
    return pl.pallas_call(
        _tc_final_body,
        grid=(_NRB,),
        in_specs=[
            pl.BlockSpec((NC, _RB, D_OUT), lambda i: (0, i, 0)),
            pl.BlockSpec((_RB, D_OUT), lambda i: (i, 0)),
            pl.BlockSpec((_RB, D_IN), lambda i: (i, 0)),
            pl.BlockSpec((1, D_OUT), lambda i: (0, 0)),
        ],
        out_specs=pl.BlockSpec((_RB, D_OUT), lambda i: (i, 0)),
        out_shape=jax.ShapeDtypeStruct((N_NODES, D_OUT), jnp.float32),
    )(acc2, ts, dis128, b2r)


def kernel(x, edge_index, W1, b1, W2, b2):
    x = x.astype(jnp.float32)
    ei = edge_index.astype(jnp.int32)
    packed = (ei[0] << 16) | ei[1]

    deg2 = _make_degree()(packed)
    dis128, xs = _tc_prescale(deg2.T, x)
    acc1 = _make_rowscatter(D_IN)(xs, packed)
    ts = _tc_mid(acc1, xs, dis128, W1, b1.reshape(1, D_HID), W2)
    acc2 = _make_rowscatter(D_OUT)(ts, packed)
    out = _tc_final(acc2, ts, dis128, b2.reshape(1, D_OUT))
    return out


# speedup vs baseline: 1.2372x; 1.0015x over previous
import functools

import jax
import jax.numpy as jnp
from jax import lax
from jax.experimental import pallas as pl
from jax.experimental.pallas import tpu as pltpu
from jax.experimental.pallas import tpu_sc as plsc

N_NODES = 10000
N_EDGES = 320000
D_IN = 128
D_HID = 256
D_OUT = 64

NC = 2
NS = 16
NW = NC * NS
NPAD = 10240
EPT = N_EDGES // NW
EB = 128
NFB = EPT // EB
TB = EPT - NFB * EB
SPT = NPAD // NS
LSPT = N_NODES - (NS - 1) * SPT

_MESH = dict(core_axis_name="c", subcore_axis_name="s", num_cores=NC,
             num_subcores=NS)


def _unpack(pk_v, base, n, isb, idb):
    for k in range(n // 16):
        w = pk_v[pl.ds(base + 16 * k, 16)]
        isb[pl.ds(16 * k, 16)] = w >> 16
        idb[pl.ds(16 * k, 16)] = w & 0xFFFF


@functools.cache
def _make_degree():
    @functools.partial(
        pl.kernel,
        out_type=jax.ShapeDtypeStruct((NC, N_NODES), jnp.float32),
        mesh=plsc.VectorSubcoreMesh(**_MESH),
        scratch_types=[
            pltpu.VMEM((EPT,), jnp.int32),
            pltpu.VMEM((EB,), jnp.int32),
            pltpu.VMEM((EB,), jnp.int32),
            pltpu.VMEM((TB,), jnp.int32),
            pltpu.VMEM((TB,), jnp.int32),
            pltpu.VMEM((EB,), jnp.float32),
            pltpu.VMEM((SPT,), jnp.float32),
            pltpu.VMEM_SHARED((NPAD,), jnp.float32),
        ],
        compiler_params=pltpu.CompilerParams(use_tc_tiling_on_sc=False),
    )
    def degree(pk_hbm, out_hbm, pk_v, isb, idb, ist, idt, ones_v, buf_v,
               acc_sh):
        c = lax.axis_index("c")
        s = lax.axis_index("s")
        tile = c * NS + s
        for i in range(EB // 16):
            ones_v[pl.ds(16 * i, 16)] = jnp.ones((16,), jnp.float32)
        for i in range(SPT // 16):
            buf_v[pl.ds(16 * i, 16)] = jnp.zeros((16,), jnp.float32)
        pltpu.sync_copy(buf_v, acc_sh.at[pl.ds(s * SPT, SPT)])
        pltpu.sync_copy(pk_hbm.at[pl.ds(tile * EPT, EPT)], pk_v)
        plsc.subcore_barrier()

        def body(b, carry):
            _unpack(pk_v, b * EB, EB, isb, idb)
            pltpu.sync_copy(ones_v, acc_sh.at[idb], add=True)
            return carry

        lax.fori_loop(0, NFB, body, 0)
        _unpack(pk_v, NFB * EB, TB, ist, idt)
        pltpu.sync_copy(ones_v.at[pl.ds(0, TB)], acc_sh.at[idt], add=True)
        plsc.subcore_barrier()

        @pl.when(s < NS - 1)
        def _():
            pltpu.sync_copy(acc_sh.at[pl.ds(s * SPT, SPT)], buf_v)
            pltpu.sync_copy(buf_v, out_hbm.at[c, pl.ds(s * SPT, SPT)])

        @pl.when(s == NS - 1)
        def _():
            pltpu.sync_copy(acc_sh.at[pl.ds(s * SPT, LSPT)],
                            buf_v.at[pl.ds(0, LSPT)])
            pltpu.sync_copy(buf_v.at[pl.ds(0, LSPT)],
                            out_hbm.at[c, pl.ds(s * SPT, LSPT)])

    return degree


@functools.cache
def _make_rowscatter(D):
    @functools.partial(
        pl.kernel,
        out_type=jax.ShapeDtypeStruct((NC, N_NODES, D), jnp.float32),
        mesh=plsc.VectorSubcoreMesh(**_MESH),
        scratch_types=[
            pltpu.VMEM((EPT,), jnp.int32),
            pltpu.VMEM((EB,), jnp.int32),
            pltpu.VMEM((EB,), jnp.int32),
            pltpu.VMEM((EB,), jnp.int32),
            pltpu.VMEM((EB,), jnp.int32),
            pltpu.VMEM((TB,), jnp.int32),
            pltpu.VMEM((TB,), jnp.int32),
            pltpu.VMEM((EB, D), jnp.float32),
            pltpu.VMEM((EB, D), jnp.float32),
            pltpu.SemaphoreType.DMA,
            pltpu.SemaphoreType.DMA,
            pltpu.VMEM_SHARED((NPAD, D), jnp.float32),
        ],
        compiler_params=pltpu.CompilerParams(use_tc_tiling_on_sc=(D == D_IN)),
    )
    def rowscatter(table_hbm, pk_hbm, out_hbm, pk_v, isb0, idb0, isb1, idb1,
                   ist, idt, rows0, rows1, gsem0, gsem1, acc_sh):
        c = lax.axis_index("c")
        s = lax.axis_index("s")
        tile = c * NS + s
        pltpu.sync_copy(pk_hbm.at[pl.ds(tile * EPT, EPT)], pk_v)
        @pl.when(s < NS - 1)
        def _():
            pltpu.sync_copy(table_hbm.at[pl.ds(s * SPT, SPT)],
                            acc_sh.at[pl.ds(s * SPT, SPT)])

        @pl.when(s == NS - 1)
        def _():
            pltpu.sync_copy(table_hbm.at[pl.ds(s * SPT, LSPT)],
                            acc_sh.at[pl.ds(s * SPT, LSPT)])

        plsc.subcore_barrier()

        _unpack(pk_v, 0, EB, isb0, idb0)
        pltpu.async_copy(table_hbm.at[isb0], rows0, gsem0)
        _unpack(pk_v, EB, EB, isb1, idb1)
        pltpu.async_copy(table_hbm.at[isb1], rows1, gsem1)

        def body(i, carry):
            b0 = 2 * i
            b1 = b0 + 1
            n0 = jnp.where(b0 + 2 >= NFB, 0, b0 + 2)
            n1 = jnp.where(b1 + 2 >= NFB, 1, b1 + 2)
            pltpu.make_async_copy(table_hbm.at[isb0], rows0, gsem0).wait()
            pltpu.sync_copy(rows0, acc_sh.at[idb0], add=True)
            _unpack(pk_v, n0 * EB, EB, isb0, idb0)
            pltpu.async_copy(table_hbm.at[isb0], rows0, gsem0)
            pltpu.make_async_copy(table_hbm.at[isb1], rows1, gsem1).wait()
            pltpu.sync_copy(rows1, acc_sh.at[idb1], add=True)
            _unpack(pk_v, n1 * EB, EB, isb1, idb1)
            pltpu.async_copy(table_hbm.at[isb1], rows1, gsem1)
            return carry

        lax.fori_loop(0, NFB // 2, body, 0)
        pltpu.make_async_copy(table_hbm.at[isb0], rows0, gsem0).wait()
        pltpu.make_async_copy(table_hbm.at[isb1], rows1, gsem1).wait()
        _unpack(pk_v, NFB * EB, TB, ist, idt)
        pltpu.sync_copy(table_hbm.at[ist], rows0.at[pl.ds(0, TB)])
        pltpu.sync_copy(rows0.at[pl.ds(0, TB)], acc_sh.at[idt], add=True)
        plsc.subcore_barrier()

        @pl.when(s < NS - 1)
        def _():
            pltpu.sync_copy(acc_sh.at[pl.ds(s * SPT, SPT)],
                            out_hbm.at[c, pl.ds(s * SPT, SPT)])

        @pl.when(s == NS - 1)
        def _():
            pltpu.sync_copy(acc_sh.at[pl.ds(s * SPT, LSPT)],
                            out_hbm.at[c, pl.ds(s * SPT, LSPT)])

    return rowscatter


def _dis(deg_ref, shape):
    deg = deg_ref[:, 0:1] + deg_ref[:, 1:2] + 1.0
    return jnp.broadcast_to(lax.rsqrt(deg), shape)


def _tc_prescale_body(deg_ref, x_ref, xs_ref):
    xs_ref[...] = x_ref[...] * _dis(deg_ref, xs_ref.shape)


_RB = 2000
_NRB = N_NODES // _RB


def _tc_prescale(deg2t, x):
    return pl.pallas_call(
        _tc_prescale_body,
        grid=(_NRB,),
        in_specs=[
            pl.BlockSpec((_RB, NC), lambda i: (i, 0)),
            pl.BlockSpec((_RB, D_IN), lambda i: (i, 0)),
        ],
        out_specs=pl.BlockSpec((_RB, D_IN), lambda i: (i, 0)),
        out_shape=jax.ShapeDtypeStruct((N_NODES, D_IN), jnp.float32),
    )(deg2t, x)


def _tc_mid_body(acc_ref, xs_ref, deg_ref, w1_ref, b1_ref, w2_ref, ts_ref):
    dis = _dis(deg_ref, (acc_ref.shape[1], D_IN))
    agg = (acc_ref[0] + acc_ref[1] - xs_ref[...]) * dis
    h = jnp.dot(agg, w1_ref[...], preferred_element_type=jnp.float32)
    h = jnp.maximum(h + b1_ref[...], 0.0)
    t = jnp.dot(h, w2_ref[...], preferred_element_type=jnp.float32)
    ts_ref[...] = t * dis[:, :D_OUT]


def _tc_mid(acc, xs, deg2t, W1, b1r, W2):
    return pl.pallas_call(
        _tc_mid_body,
        grid=(_NRB,),
        in_specs=[
            pl.BlockSpec((NC, _RB, D_IN), lambda i: (0, i, 0)),
            pl.BlockSpec((_RB, D_IN), lambda i: (i, 0)),
            pl.BlockSpec((_RB, NC), lambda i: (i, 0)),
            pl.BlockSpec((D_IN, D_HID), lambda i: (0, 0)),
            pl.BlockSpec((1, D_HID), lambda i: (0, 0)),
            pl.BlockSpec((D_HID, D_OUT), lambda i: (0, 0)),
        ],
        out_specs=pl.BlockSpec((_RB, D_OUT), lambda i: (i, 0)),
        out_shape=jax.ShapeDtypeStruct((N_NODES, D_OUT), jnp.float32),
    )(acc, xs, deg2t, W1, b1r, W2)


def _tc_final_body(acc_ref, ts_ref, deg_ref, b2_ref, out_ref):
    dis = _dis(deg_ref, out_ref.shape)
    out_ref[...] = (acc_ref[0] + acc_ref[1] - ts_ref[...]) * dis + b2_ref[...]


def _tc_final(acc2, ts, deg2t, b2r):
    return pl.pallas_call(
        _tc_final_body,
        grid=(_NRB,),
        in_specs=[
            pl.BlockSpec((NC, _RB, D_OUT), lambda i: (0, i, 0)),
            pl.BlockSpec((_RB, D_OUT), lambda i: (i, 0)),
            pl.BlockSpec((_RB, NC), lambda i: (i, 0)),
            pl.BlockSpec((1, D_OUT), lambda i: (0, 0)),
        ],
        out_specs=pl.BlockSpec((_RB, D_OUT), lambda i: (i, 0)),
        out_shape=jax.ShapeDtypeStruct((N_NODES, D_OUT), jnp.float32),
    )(acc2, ts, deg2t, b2r)


def kernel(x, edge_index, W1, b1, W2, b2):
    x = x.astype(jnp.float32)
    ei = edge_index.astype(jnp.int32)
    packed = (ei[0] << 16) | ei[1]

    deg2 = _make_degree()(packed)
    deg2t = deg2.T
    xs = _tc_prescale(deg2t, x)
    acc1 = _make_rowscatter(D_IN)(xs, packed)
    ts = _tc_mid(acc1, xs, deg2t, W1, b1.reshape(1, D_HID), W2)
    acc2 = _make_rowscatter(D_OUT)(ts, packed)
    out = _tc_final(acc2, ts, deg2t, b2.reshape(1, D_OUT))
    return out


# speedup vs baseline: 1.3017x; 1.0522x over previous
import functools

import jax
import jax.numpy as jnp
from jax import lax
from jax.experimental import pallas as pl
from jax.experimental.pallas import tpu as pltpu
from jax.experimental.pallas import tpu_sc as plsc

N_NODES = 10000
N_EDGES = 320000
D_IN = 128
D_HID = 256
D_OUT = 64

NC = 2
NS = 16
NW = NC * NS
NPAD = 10240
EPT = N_EDGES // NW
EB = 128
NFB = EPT // EB
TB = EPT - NFB * EB
SPT = NPAD // NS
LSPT = N_NODES - (NS - 1) * SPT

_MESH = dict(core_axis_name="c", subcore_axis_name="s", num_cores=NC,
             num_subcores=NS)


def _unpack(pk_v, base, n, isb, idb):
    for k in range(n // 16):
        w = pk_v[pl.ds(base + 16 * k, 16)]
        isb[pl.ds(16 * k, 16)] = w >> 16
        idb[pl.ds(16 * k, 16)] = w & 0xFFFF


@functools.cache
def _make_degree():
    @functools.partial(
        pl.kernel,
        out_type=jax.ShapeDtypeStruct((NC, N_NODES), jnp.float32),
        mesh=plsc.VectorSubcoreMesh(**_MESH),
        scratch_types=[
            pltpu.VMEM((EPT,), jnp.int32),
            pltpu.VMEM((EB,), jnp.int32),
            pltpu.VMEM((EB,), jnp.int32),
            pltpu.VMEM((TB,), jnp.int32),
            pltpu.VMEM((TB,), jnp.int32),
            pltpu.VMEM((EB,), jnp.float32),
            pltpu.VMEM((SPT,), jnp.float32),
            pltpu.VMEM_SHARED((NPAD,), jnp.float32),
        ],
        compiler_params=pltpu.CompilerParams(use_tc_tiling_on_sc=False),
    )
    def degree(pk_hbm, out_hbm, pk_v, isb, idb, ist, idt, ones_v, buf_v,
               acc_sh):
        c = lax.axis_index("c")
        s = lax.axis_index("s")
        tile = c * NS + s
        for i in range(EB // 16):
            ones_v[pl.ds(16 * i, 16)] = jnp.ones((16,), jnp.float32)
        for i in range(SPT // 16):
            buf_v[pl.ds(16 * i, 16)] = jnp.zeros((16,), jnp.float32)
        pltpu.sync_copy(buf_v, acc_sh.at[pl.ds(s * SPT, SPT)])
        pltpu.sync_copy(pk_hbm.at[pl.ds(tile * EPT, EPT)], pk_v)
        plsc.subcore_barrier()

        def body(b, carry):
            _unpack(pk_v, b * EB, EB, isb, idb)
            pltpu.sync_copy(ones_v, acc_sh.at[idb], add=True)
            return carry

        lax.fori_loop(0, NFB, body, 0)
        _unpack(pk_v, NFB * EB, TB, ist, idt)
        pltpu.sync_copy(ones_v.at[pl.ds(0, TB)], acc_sh.at[idt], add=True)
        plsc.subcore_barrier()

        @pl.when(s < NS - 1)
        def _():
            pltpu.sync_copy(acc_sh.at[pl.ds(s * SPT, SPT)], buf_v)
            pltpu.sync_copy(buf_v, out_hbm.at[c, pl.ds(s * SPT, SPT)])

        @pl.when(s == NS - 1)
        def _():
            pltpu.sync_copy(acc_sh.at[pl.ds(s * SPT, LSPT)],
                            buf_v.at[pl.ds(0, LSPT)])
            pltpu.sync_copy(buf_v.at[pl.ds(0, LSPT)],
                            out_hbm.at[c, pl.ds(s * SPT, LSPT)])

    return degree


@functools.cache
def _make_rowscatter(D):
    nbuf = 2 if D == D_IN else 3
    scratch = [pltpu.VMEM((EPT,), jnp.int32)]
    for _ in range(nbuf):
        scratch += [pltpu.VMEM((EB,), jnp.int32), pltpu.VMEM((EB,), jnp.int32)]
    scratch += [pltpu.VMEM((TB,), jnp.int32), pltpu.VMEM((TB,), jnp.int32)]
    scratch += [pltpu.VMEM((EB, D), jnp.float32)] * nbuf
    scratch += [pltpu.SemaphoreType.DMA] * nbuf
    scratch += [pltpu.VMEM_SHARED((NPAD, D), jnp.float32)]

    @functools.partial(
        pl.kernel,
        out_type=jax.ShapeDtypeStruct((NC, N_NODES, D), jnp.float32),
        mesh=plsc.VectorSubcoreMesh(**_MESH),
        scratch_types=scratch,
        compiler_params=pltpu.CompilerParams(use_tc_tiling_on_sc=(D == D_IN)),
    )
    def rowscatter(table_hbm, pk_hbm, out_hbm, pk_v, *refs):
        isb = [refs[2 * j] for j in range(nbuf)]
        idb = [refs[2 * j + 1] for j in range(nbuf)]
        ist, idt = refs[2 * nbuf], refs[2 * nbuf + 1]
        rows = list(refs[2 * nbuf + 2:3 * nbuf + 2])
        gsem = list(refs[3 * nbuf + 2:4 * nbuf + 2])
        acc_sh = refs[-1]
        c = lax.axis_index("c")
        s = lax.axis_index("s")
        tile = c * NS + s
        pltpu.sync_copy(pk_hbm.at[pl.ds(tile * EPT, EPT)], pk_v)
        @pl.when(s < NS - 1)
        def _():
            pltpu.sync_copy(table_hbm.at[pl.ds(s * SPT, SPT)],
                            acc_sh.at[pl.ds(s * SPT, SPT)])

        @pl.when(s == NS - 1)
        def _():
            pltpu.sync_copy(table_hbm.at[pl.ds(s * SPT, LSPT)],
                            acc_sh.at[pl.ds(s * SPT, LSPT)])

        plsc.subcore_barrier()

        for j in range(nbuf):
            _unpack(pk_v, j * EB, EB, isb[j], idb[j])
            pltpu.async_copy(table_hbm.at[isb[j]], rows[j], gsem[j])

        def body(i, carry):
            for j in range(nbuf):
                b = nbuf * i + j
                n = jnp.where(b + nbuf >= NFB, j, b + nbuf)
                pltpu.make_async_copy(table_hbm.at[isb[j]], rows[j],
                                      gsem[j]).wait()
                pltpu.sync_copy(rows[j], acc_sh.at[idb[j]], add=True)
                _unpack(pk_v, n * EB, EB, isb[j], idb[j])
                pltpu.async_copy(table_hbm.at[isb[j]], rows[j], gsem[j])
            return carry

        lax.fori_loop(0, NFB // nbuf, body, 0)
        for j in range(nbuf):
            pltpu.make_async_copy(table_hbm.at[isb[j]], rows[j],
                                  gsem[j]).wait()
        rows0 = rows[0]
        _unpack(pk_v, NFB * EB, TB, ist, idt)
        pltpu.sync_copy(table_hbm.at[ist], rows0.at[pl.ds(0, TB)])
        pltpu.sync_copy(rows0.at[pl.ds(0, TB)], acc_sh.at[idt], add=True)
        plsc.subcore_barrier()

        @pl.when(s < NS - 1)
        def _():
            pltpu.sync_copy(acc_sh.at[pl.ds(s * SPT, SPT)],
                            out_hbm.at[c, pl.ds(s * SPT, SPT)])

        @pl.when(s == NS - 1)
        def _():
            pltpu.sync_copy(acc_sh.at[pl.ds(s * SPT, LSPT)],
                            out_hbm.at[c, pl.ds(s * SPT, LSPT)])

    return rowscatter


def _dis(deg_ref, shape):
    deg = deg_ref[:, 0:1] + deg_ref[:, 1:2] + 1.0
    return jnp.broadcast_to(lax.rsqrt(deg), shape)


def _tc_prescale_body(deg_ref, x_ref, xs_ref):
    xs_ref[...] = x_ref[...] * _dis(deg_ref, xs_ref.shape)


_RB = 2000
_NRB = N_NODES // _RB


def _tc_prescale(deg2t, x):
    return pl.pallas_call(
        _tc_prescale_body,
        grid=(_NRB,),
        in_specs=[
            pl.BlockSpec((_RB, NC), lambda i: (i, 0)),
            pl.BlockSpec((_RB, D_IN), lambda i: (i, 0)),
        ],
        out_specs=pl.BlockSpec((_RB, D_IN), lambda i: (i, 0)),
        out_shape=jax.ShapeDtypeStruct((N_NODES, D_IN), jnp.float32),
    )(deg2t, x)


def _tc_mid_body(acc_ref, xs_ref, deg_ref, w1_ref, b1_ref, w2_ref, ts_ref):
    dis = _dis(deg_ref, (acc_ref.shape[1], D_IN))
    agg = (acc_ref[0] + acc_ref[1] - xs_ref[...]) * dis
    h = jnp.dot(agg, w1_ref[...], preferred_element_type=jnp.float32)
    h = jnp.maximum(h + b1_ref[...], 0.0)
    t = jnp.dot(h, w2_ref[...], preferred_element_type=jnp.float32)
    ts_ref[...] = t * dis[:, :D_OUT]


def _tc_mid(acc, xs, deg2t, W1, b1r, W2):
    return pl.pallas_call(
        _tc_mid_body,
        grid=(_NRB,),
        in_specs=[
            pl.BlockSpec((NC, _RB, D_IN), lambda i: (0, i, 0)),
            pl.BlockSpec((_RB, D_IN), lambda i: (i, 0)),
            pl.BlockSpec((_RB, NC), lambda i: (i, 0)),
            pl.BlockSpec((D_IN, D_HID), lambda i: (0, 0)),
            pl.BlockSpec((1, D_HID), lambda i: (0, 0)),
            pl.BlockSpec((D_HID, D_OUT), lambda i: (0, 0)),
        ],
        out_specs=pl.BlockSpec((_RB, D_OUT), lambda i: (i, 0)),
        out_shape=jax.ShapeDtypeStruct((N_NODES, D_OUT), jnp.float32),
    )(acc, xs, deg2t, W1, b1r, W2)


def _tc_final_body(acc_ref, ts_ref, deg_ref, b2_ref, out_ref):
    dis = _dis(deg_ref, out_ref.shape)
    out_ref[...] = (acc_ref[0] + acc_ref[1] - ts_ref[...]) * dis + b2_ref[...]


def _tc_final(acc2, ts, deg2t, b2r):
    return pl.pallas_call(
        _tc_final_body,
        grid=(_NRB,),
        in_specs=[
            pl.BlockSpec((NC, _RB, D_OUT), lambda i: (0, i, 0)),
            pl.BlockSpec((_RB, D_OUT), lambda i: (i, 0)),
            pl.BlockSpec((_RB, NC), lambda i: (i, 0)),
            pl.BlockSpec((1, D_OUT), lambda i: (0, 0)),
        ],
        out_specs=pl.BlockSpec((_RB, D_OUT), lambda i: (i, 0)),
        out_shape=jax.ShapeDtypeStruct((N_NODES, D_OUT), jnp.float32),
    )(acc2, ts, deg2t, b2r)


def kernel(x, edge_index, W1, b1, W2, b2):
    x = x.astype(jnp.float32)
    ei = edge_index.astype(jnp.int32)
    packed = (ei[0] << 16) | ei[1]

    deg2 = _make_degree()(packed)
    deg2t = deg2.T
    xs = _tc_prescale(deg2t, x)
    acc1 = _make_rowscatter(D_IN)(xs, packed)
    ts = _tc_mid(acc1, xs, deg2t, W1, b1.reshape(1, D_HID), W2)
    acc2 = _make_rowscatter(D_OUT)(ts, packed)
    out = _tc_final(acc2, ts, deg2t, b2.reshape(1, D_OUT))
    return out
